# initial kernel scaffold (unmeasured)
import jax
import jax.numpy as jnp
from jax import lax
from jax.experimental import pallas as pl
from jax.experimental.pallas import tpu as pltpu

N_DEV = 4
K_CHUNK = 1024
N_BLK = 1024
M_PER = 2048


def kernel(x, w_mat):
    m_per, k_dim = x.shape
    _, n_dim = w_mat.shape
    n_blk = n_dim // N_DEV
    n_k = k_dim // K_CHUNK

    def body(x_hbm, w_hbm, out_hbm, x_buf, w_buf, acc, stage, send_buf,
             recv_buf, x_sems, w_sems, out_sems, send_sems, recv_sems,
             exit_sem):
        my = lax.axis_index("i")

        barrier_sem = pltpu.get_barrier_semaphore()
        for p in range(1, N_DEV):
            pl.semaphore_signal(
                barrier_sem, inc=1,
                device_id=((my + p) % N_DEV,),
                device_id_type=pl.DeviceIdType.MESH,
            )
        pl.semaphore_wait(barrier_sem, N_DEV - 1)

        offsets = [1, 2, 3, 0]
        steps = [(s, k) for s in range(N_DEV) for k in range(n_k)]

        inflight = [None, None]

        def issue_fetch(i):
            s, k = steps[i]
            d = (my + offsets[s]) % N_DEV
            slot = i % 2
            cx = pltpu.make_async_copy(
                x_hbm.at[:, pl.ds(k * K_CHUNK, K_CHUNK)],
                x_buf.at[slot],
                x_sems.at[slot],
            )
            cw = pltpu.make_async_copy(
                w_hbm.at[pl.ds(k * K_CHUNK, K_CHUNK), pl.ds(d * n_blk, n_blk)],
                w_buf.at[slot],
                w_sems.at[slot],
            )
            cx.start()
            cw.start()
            inflight[slot] = (cx, cw)

        out_dmas = []

        issue_fetch(0)
        for i, (s, k) in enumerate(steps):
            slot = i % 2
            if i + 1 < len(steps):
                issue_fetch(i + 1)
            cx, cw = inflight[slot]
            cx.wait()
            cw.wait()
            prod = jnp.dot(
                x_buf[slot], w_buf[slot], preferred_element_type=jnp.float32
            )
            if k == 0:
                acc[...] = prod
            else:
                acc[...] += prod

            if k == n_k - 1:
                off = offsets[s]
                if off == 0:
                    stage[...] = jnp.maximum(acc[...], 0.0)
                    cp = pltpu.make_async_copy(
                        stage,
                        out_hbm.at[pl.ds(my * M_PER, M_PER), :],
                        out_sems.at[0],
                    )
                    cp.start()
                    out_dmas.append(cp)
                else:
                    t = s
                    d = (my + off) % N_DEV
                    send_buf[t] = jnp.maximum(acc[...], 0.0).astype(
                        jnp.bfloat16
                    )
                    rdma = pltpu.make_async_remote_copy(
                        src_ref=send_buf.at[t],
                        dst_ref=recv_buf.at[t],
                        send_sem=send_sems.at[t],
                        recv_sem=recv_sems.at[t],
                        device_id=(d,),
                        device_id_type=pl.DeviceIdType.MESH,
                    )
                    rdma.start()
                    out_dmas.append(rdma)

        own_dma = out_dmas[-1]
        send_rdmas = [h for h in out_dmas[:-1]]
        for t in range(N_DEV - 1):
            recv = pltpu.make_async_remote_copy(
                src_ref=send_buf.at[t],
                dst_ref=recv_buf.at[t],
                send_sem=send_sems.at[t],
                recv_sem=recv_sems.at[t],
                device_id=((my - t - 1) % N_DEV,),
                device_id_type=pl.DeviceIdType.MESH,
            )
            recv.wait_recv()
            own_dma.wait() if t == 0 else prev_cp.wait()
            j = (my - t - 1) % N_DEV
            stage[...] = recv_buf[t].astype(jnp.float32)
            prev_cp = pltpu.make_async_copy(
                stage,
                out_hbm.at[pl.ds(j * M_PER, M_PER), :],
                out_sems.at[(t + 1) % 2],
            )
            prev_cp.start()
        prev_cp.wait()
        for rdma in send_rdmas:
            rdma.wait_send()

        for p in range(1, N_DEV):
            pl.semaphore_signal(
                exit_sem, inc=1,
                device_id=((my + p) % N_DEV,),
                device_id_type=pl.DeviceIdType.MESH,
            )
        pl.semaphore_wait(exit_sem, N_DEV - 1)

    return pl.pallas_call(
        body,
        out_shape=jax.ShapeDtypeStruct((N_DEV * m_per, n_blk), jnp.float32),
        in_specs=[
            pl.BlockSpec(memory_space=pltpu.ANY),
            pl.BlockSpec(memory_space=pltpu.ANY),
        ],
        out_specs=pl.BlockSpec(memory_space=pltpu.ANY),
        scratch_shapes=[
            pltpu.VMEM((2, m_per, K_CHUNK), jnp.bfloat16),
            pltpu.VMEM((2, K_CHUNK, n_blk), jnp.bfloat16),
            pltpu.VMEM((m_per, n_blk), jnp.float32),
            pltpu.VMEM((m_per, n_blk), jnp.float32),
            pltpu.VMEM((N_DEV - 1, m_per, n_blk), jnp.bfloat16),
            pltpu.VMEM((N_DEV - 1, m_per, n_blk), jnp.bfloat16),
            pltpu.SemaphoreType.DMA((2,)),
            pltpu.SemaphoreType.DMA((2,)),
            pltpu.SemaphoreType.DMA((2,)),
            pltpu.SemaphoreType.DMA((N_DEV - 1,)),
            pltpu.SemaphoreType.DMA((N_DEV - 1,)),
            pltpu.SemaphoreType.REGULAR,
        ],
        compiler_params=pltpu.CompilerParams(
            collective_id=0,
            vmem_limit_bytes=100 * 1024 * 1024,
        ),
    )(x, w_mat)


# baseline (device time: 264107 ns/iter reference)
import jax
import jax.numpy as jnp
from jax import lax
from jax.experimental import pallas as pl
from jax.experimental.pallas import tpu as pltpu

N_DEV = 4
K_CHUNK = 1024
M_TILE = 256


def kernel(x, w_mat):
    x = x.astype(jnp.bfloat16)
    m_per, k_dim = x.shape
    _, n_dim = w_mat.shape
    n_blk = n_dim // N_DEV
    n_k = k_dim // K_CHUNK
    n_m = m_per // M_TILE

    def body(x_hbm, w_hbm, out_hbm, recv_hbm, x_tile, w_stage, w_col, res,
             send_buf, x_sems, w_sem, out_sem, send_sems, recv_sems,
             exit_sem):
        my = lax.axis_index("i")

        barrier_sem = pltpu.get_barrier_semaphore()
        for p in range(1, N_DEV):
            pl.semaphore_signal(
                barrier_sem, inc=1,
                device_id=((my + p) % N_DEV,),
                device_id_type=pl.DeviceIdType.MESH,
            )
        pl.semaphore_wait(barrier_sem, N_DEV - 1)

        def start_x(m, slot):
            pltpu.make_async_copy(
                x_hbm.at[pl.ds(m * M_TILE, M_TILE), :],
                x_tile.at[slot],
                x_sems.at[slot],
            ).start()

        def wait_x(m, slot):
            pltpu.make_async_copy(
                x_hbm.at[pl.ds(m * M_TILE, M_TILE), :],
                x_tile.at[slot],
                x_sems.at[slot],
            ).wait()

        def start_w(d, k):
            pltpu.make_async_copy(
                w_hbm.at[pl.ds(k * K_CHUNK, K_CHUNK), pl.ds(d * n_blk, n_blk)],
                w_stage,
                w_sem,
            ).start()

        def wait_w(d, k):
            pltpu.make_async_copy(
                w_hbm.at[pl.ds(k * K_CHUNK, K_CHUNK), pl.ds(d * n_blk, n_blk)],
                w_stage,
                w_sem,
            ).wait()

        def convert_w(col_slot, k):
            w_col[col_slot, pl.ds(k * K_CHUNK, K_CHUNK), :] = w_stage[
                ...
            ].astype(jnp.bfloat16)

        offsets = [1, 2, 3, 0]

        start_x(0, 0)
        start_x(1, 1)
        d0 = (my + offsets[0]) % N_DEV
        start_w(d0, 0)
        for k in range(n_k):
            wait_w(d0, k)
            convert_w(0, k)
            if k + 1 < n_k:
                start_w(d0, k + 1)

        send_rdmas = []

        for s in range(N_DEV):
            cur = s % 2
            nxt = (s + 1) % 2
            off = offsets[s]
            d = (my + off) % N_DEV
            remote = off != 0
            d_next = (my + offsets[s + 1]) % N_DEV if s + 1 < N_DEV else None

            if remote:
                dst_slot = s % 2
                if s == 2:
                    send_rdmas[0].wait_send()
            else:
                send_rdmas[1].wait_send()
                dst_slot = 1

            if s + 1 < N_DEV:
                start_w(d_next, 0)

            def pair_body(mm, _, cur=cur, nxt=nxt, s=s, dst_slot=dst_slot,
                          d_next=d_next):
                m0 = 2 * mm
                m1 = m0 + 1
                wait_x(m0, 0)
                r0 = jnp.dot(
                    x_tile[0], w_col[cur], preferred_element_type=jnp.float32
                )
                send_buf[dst_slot, pl.ds(m0 * M_TILE, M_TILE), :] = (
                    jnp.maximum(r0, 0.0).astype(jnp.bfloat16)
                )

                @pl.when(m0 + 2 < n_m)
                def _():
                    start_x(m0 + 2, 0)

                if s + 1 < N_DEV:
                    k0 = 2 * mm
                    wait_w(d_next, k0)
                    convert_w(nxt, k0)

                    @pl.when(k0 + 1 < n_k)
                    def _():
                        start_w(d_next, k0 + 1)

                wait_x(m1, 1)
                r1 = jnp.dot(
                    x_tile[1], w_col[cur], preferred_element_type=jnp.float32
                )
                send_buf[dst_slot, pl.ds(m1 * M_TILE, M_TILE), :] = (
                    jnp.maximum(r1, 0.0).astype(jnp.bfloat16)
                )

                @pl.when(m1 + 2 < n_m)
                def _():
                    start_x(m1 + 2, 1)

                if s + 1 < N_DEV:
                    k1 = 2 * mm + 1
                    wait_w(d_next, k1)
                    convert_w(nxt, k1)

                    @pl.when(k1 + 1 < n_k)
                    def _():
                        start_w(d_next, k1 + 1)

                return 0

            lax.fori_loop(0, n_m // 2, pair_body, 0)

            if s + 1 < N_DEV:
                start_x(0, 0)
                start_x(1, 1)

            if remote:
                t = s
                rdma = pltpu.make_async_remote_copy(
                    src_ref=send_buf.at[dst_slot],
                    dst_ref=recv_hbm.at[t],
                    send_sem=send_sems.at[dst_slot],
                    recv_sem=recv_sems.at[t],
                    device_id=(d,),
                    device_id_type=pl.DeviceIdType.MESH,
                )
                rdma.start()
                send_rdmas.append(rdma)

        res[...] = send_buf[1].astype(jnp.float32)
        own_cp = pltpu.make_async_copy(
            res, out_hbm.at[pl.ds(my * m_per, m_per), :], out_sem
        )
        own_cp.start()

        send_rdmas[2].wait_send()
        prev_cp = own_cp
        for t in range(N_DEV - 1):
            recv = pltpu.make_async_remote_copy(
                src_ref=send_buf.at[0],
                dst_ref=recv_hbm.at[t],
                send_sem=send_sems.at[0],
                recv_sem=recv_sems.at[t],
                device_id=((my - t - 1) % N_DEV,),
                device_id_type=pl.DeviceIdType.MESH,
            )
            recv.wait_recv()
            bounce = pltpu.make_async_copy(
                recv_hbm.at[t], send_buf.at[0], x_sems.at[0]
            )
            bounce.start()
            bounce.wait()
            prev_cp.wait()
            j = (my - t - 1) % N_DEV
            res[...] = send_buf[0].astype(jnp.float32)
            prev_cp = pltpu.make_async_copy(
                res, out_hbm.at[pl.ds(j * m_per, m_per), :], out_sem
            )
            prev_cp.start()
        prev_cp.wait()

        for p in range(1, N_DEV):
            pl.semaphore_signal(
                exit_sem, inc=1,
                device_id=((my + p) % N_DEV,),
                device_id_type=pl.DeviceIdType.MESH,
            )
        pl.semaphore_wait(exit_sem, N_DEV - 1)

    out, _recv = pl.pallas_call(
        body,
        out_shape=(
            jax.ShapeDtypeStruct((N_DEV * m_per, n_blk), jnp.float32),
            jax.ShapeDtypeStruct((N_DEV - 1, m_per, n_blk), jnp.bfloat16),
        ),
        in_specs=[
            pl.BlockSpec(memory_space=pl.ANY),
            pl.BlockSpec(memory_space=pl.ANY),
        ],
        out_specs=(
            pl.BlockSpec(memory_space=pl.ANY),
            pl.BlockSpec(memory_space=pl.ANY),
        ),
        scratch_shapes=[
            pltpu.VMEM((2, M_TILE, k_dim), jnp.bfloat16),
            pltpu.VMEM((K_CHUNK, n_blk), jnp.float32),
            pltpu.VMEM((2, k_dim, n_blk), jnp.bfloat16),
            pltpu.VMEM((m_per, n_blk), jnp.float32),
            pltpu.VMEM((2, m_per, n_blk), jnp.bfloat16),
            pltpu.SemaphoreType.DMA((2,)),
            pltpu.SemaphoreType.DMA,
            pltpu.SemaphoreType.DMA,
            pltpu.SemaphoreType.DMA((2,)),
            pltpu.SemaphoreType.DMA((N_DEV - 1,)),
            pltpu.SemaphoreType.REGULAR,
        ],
        compiler_params=pltpu.CompilerParams(
            collective_id=0,
            vmem_limit_bytes=100 * 1024 * 1024,
        ),
    )(x, w_mat)
    return out


# device time: 254097 ns/iter; 1.0394x vs baseline; 1.0394x over previous
import jax
import jax.numpy as jnp
from jax import lax
from jax.experimental import pallas as pl
from jax.experimental.pallas import tpu as pltpu

N_DEV = 4
K_CHUNK = 1024
N_BLK = 1024
M_PER = 2048


def kernel(x, w_mat):
    x = x.astype(jnp.bfloat16)
    m_per, k_dim = x.shape
    _, n_dim = w_mat.shape
    n_blk = n_dim // N_DEV
    n_k = k_dim // K_CHUNK

    def body(x_hbm, w_hbm, out_hbm, x_buf, w_stage, w_buf, acc, send_buf,
             recv_buf, x_sems, w_sems, out_sems, send_sems, recv_sems,
             exit_sem):
        my = lax.axis_index("i")

        barrier_sem = pltpu.get_barrier_semaphore()
        for p in range(1, N_DEV):
            pl.semaphore_signal(
                barrier_sem, inc=1,
                device_id=((my + p) % N_DEV,),
                device_id_type=pl.DeviceIdType.MESH,
            )
        pl.semaphore_wait(barrier_sem, N_DEV - 1)

        offsets = [1, 2, 3, 0]

        def fetch(k, d, slot):
            cx = pltpu.make_async_copy(
                x_hbm.at[:, pl.ds(k * K_CHUNK, K_CHUNK)],
                x_buf.at[slot],
                x_sems.at[slot],
            )
            cw = pltpu.make_async_copy(
                w_hbm.at[pl.ds(k * K_CHUNK, K_CHUNK), pl.ds(d * n_blk, n_blk)],
                w_stage.at[slot],
                w_sems.at[slot],
            )
            cx.start()
            cw.start()
            return cx, cw

        def wait_fetch(k, d, slot):
            pltpu.make_async_copy(
                x_hbm.at[:, pl.ds(k * K_CHUNK, K_CHUNK)],
                x_buf.at[slot],
                x_sems.at[slot],
            ).wait()
            pltpu.make_async_copy(
                w_hbm.at[pl.ds(k * K_CHUNK, K_CHUNK), pl.ds(d * n_blk, n_blk)],
                w_stage.at[slot],
                w_sems.at[slot],
            ).wait()
            w_buf[slot] = w_stage[slot].astype(jnp.bfloat16)

        send_rdmas = []
        own_dma = None

        d0 = (my + offsets[0]) % N_DEV
        fetch(0, d0, 0)
        fetch(1, d0, 1)

        for s in range(N_DEV):
            d = (my + offsets[s]) % N_DEV

            def kbody(kk, _, d=d):
                k0 = 2 * kk
                k1 = k0 + 1
                wait_fetch(k0, d, 0)
                prod0 = jnp.dot(
                    x_buf[0], w_buf[0], preferred_element_type=jnp.float32
                )

                @pl.when(kk == 0)
                def _():
                    acc[...] = prod0

                @pl.when(kk > 0)
                def _():
                    acc[...] += prod0

                @pl.when(k0 + 2 < n_k)
                def _():
                    fetch(k0 + 2, d, 0)

                wait_fetch(k1, d, 1)
                acc[...] += jnp.dot(
                    x_buf[1], w_buf[1], preferred_element_type=jnp.float32
                )

                @pl.when(k1 + 2 < n_k)
                def _():
                    fetch(k1 + 2, d, 1)

                return 0

            lax.fori_loop(0, n_k // 2, kbody, 0)

            if s + 1 < N_DEV:
                d_next = (my + offsets[s + 1]) % N_DEV
                fetch(0, d_next, 0)
                fetch(1, d_next, 1)

            off = offsets[s]
            if off == 0:
                acc[...] = jnp.maximum(acc[...], 0.0)
                own_dma = pltpu.make_async_copy(
                    acc,
                    out_hbm.at[pl.ds(my * M_PER, M_PER), :],
                    out_sems.at[0],
                )
                own_dma.start()
            else:
                t = s
                sslot = s % 2
                if s == 2:
                    send_rdmas[0].wait_send()
                send_buf[sslot] = jnp.maximum(acc[...], 0.0).astype(
                    jnp.bfloat16
                )
                rdma = pltpu.make_async_remote_copy(
                    src_ref=send_buf.at[sslot],
                    dst_ref=recv_buf.at[t],
                    send_sem=send_sems.at[sslot],
                    recv_sem=recv_sems.at[t],
                    device_id=(d,),
                    device_id_type=pl.DeviceIdType.MESH,
                )
                rdma.start()
                send_rdmas.append(rdma)

        for t in range(N_DEV - 1):
            recv = pltpu.make_async_remote_copy(
                src_ref=send_buf.at[t % 2],
                dst_ref=recv_buf.at[t],
                send_sem=send_sems.at[t % 2],
                recv_sem=recv_sems.at[t],
                device_id=((my - t - 1) % N_DEV,),
                device_id_type=pl.DeviceIdType.MESH,
            )
            recv.wait_recv()
            own_dma.wait() if t == 0 else prev_cp.wait()
            j = (my - t - 1) % N_DEV
            acc[...] = recv_buf[t].astype(jnp.float32)
            prev_cp = pltpu.make_async_copy(
                acc,
                out_hbm.at[pl.ds(j * M_PER, M_PER), :],
                out_sems.at[(t + 1) % 2],
            )
            prev_cp.start()
        prev_cp.wait()
        for rdma in send_rdmas[1:]:
            rdma.wait_send()

        for p in range(1, N_DEV):
            pl.semaphore_signal(
                exit_sem, inc=1,
                device_id=((my + p) % N_DEV,),
                device_id_type=pl.DeviceIdType.MESH,
            )
        pl.semaphore_wait(exit_sem, N_DEV - 1)

    return pl.pallas_call(
        body,
        out_shape=jax.ShapeDtypeStruct((N_DEV * m_per, n_blk), jnp.float32),
        in_specs=[
            pl.BlockSpec(memory_space=pltpu.ANY),
            pl.BlockSpec(memory_space=pltpu.ANY),
        ],
        out_specs=pl.BlockSpec(memory_space=pltpu.ANY),
        scratch_shapes=[
            pltpu.VMEM((2, m_per, K_CHUNK), jnp.bfloat16),
            pltpu.VMEM((2, K_CHUNK, n_blk), jnp.float32),
            pltpu.VMEM((2, K_CHUNK, n_blk), jnp.bfloat16),
            pltpu.VMEM((m_per, n_blk), jnp.float32),
            pltpu.VMEM((2, m_per, n_blk), jnp.bfloat16),
            pltpu.VMEM((N_DEV - 1, m_per, n_blk), jnp.bfloat16),
            pltpu.SemaphoreType.DMA((2,)),
            pltpu.SemaphoreType.DMA((2,)),
            pltpu.SemaphoreType.DMA((2,)),
            pltpu.SemaphoreType.DMA((2,)),
            pltpu.SemaphoreType.DMA((N_DEV - 1,)),
            pltpu.SemaphoreType.REGULAR,
        ],
        compiler_params=pltpu.CompilerParams(
            collective_id=0,
            vmem_limit_bytes=100 * 1024 * 1024,
        ),
    )(x, w_mat)


# device time: 227871 ns/iter; 1.1590x vs baseline; 1.1151x over previous
import jax
import jax.numpy as jnp
from jax import lax
from jax.experimental import pallas as pl
from jax.experimental.pallas import tpu as pltpu

N_DEV = 4
K_CHUNK = 1024
N_BLK = 1024
M_PER = 2048


def kernel(x, w_mat):
    m_per, k_dim = x.shape
    _, n_dim = w_mat.shape
    n_blk = n_dim // N_DEV
    n_k = k_dim // K_CHUNK

    def body(x_hbm, w_hbm, out_hbm, xbf_hbm, x_buf, x_stage, w_stage, w_buf,
             acc, send_buf, recv_buf, x_sems, xs_sem, wb_sems, w_sems,
             out_sems, send_sems, recv_sems, exit_sem):
        my = lax.axis_index("i")

        barrier_sem = pltpu.get_barrier_semaphore()
        for p in range(1, N_DEV):
            pl.semaphore_signal(
                barrier_sem, inc=1,
                device_id=((my + p) % N_DEV,),
                device_id_type=pl.DeviceIdType.MESH,
            )
        pl.semaphore_wait(barrier_sem, N_DEV - 1)

        offsets = [1, 2, 3, 0]

        def w_copy(k, d, slot):
            return pltpu.make_async_copy(
                w_hbm.at[pl.ds(k * K_CHUNK, K_CHUNK), pl.ds(d * n_blk, n_blk)],
                w_stage.at[slot],
                w_sems.at[slot],
            )

        def wait_w_convert(k, d, slot):
            w_copy(k, d, slot).wait()
            w_buf[slot] = w_stage[slot].astype(jnp.bfloat16)

        def xf32_copy(k):
            return pltpu.make_async_copy(
                x_hbm.at[:, pl.ds(k * K_CHUNK, K_CHUNK)], x_stage, xs_sem
            )

        def xbf_copy(k, slot):
            return pltpu.make_async_copy(
                xbf_hbm.at[:, pl.ds(k * K_CHUNK, K_CHUNK)],
                x_buf.at[slot],
                x_sems.at[slot],
            )

        def wb_copy(k, slot):
            return pltpu.make_async_copy(
                x_buf.at[slot],
                xbf_hbm.at[:, pl.ds(k * K_CHUNK, K_CHUNK)],
                wb_sems.at[slot],
            )

        send_rdmas = []
        own_dma = None

        d0 = (my + offsets[0]) % N_DEV
        xf32_copy(0).start()
        w_copy(0, d0, 0).start()
        w_copy(1, d0, 1).start()

        def kbody0(kk, _):
            k0 = 2 * kk
            k1 = k0 + 1
            xf32_copy(k0).wait()

            @pl.when(k0 >= 2)
            def _():
                wb_copy(k0 - 2, 0).wait()

            x_buf[0] = x_stage[...].astype(jnp.bfloat16)
            xf32_copy(k0 + 1).start()
            wb_copy(k0, 0).start()
            wait_w_convert(k0, d0, 0)
            prod0 = jnp.dot(
                x_buf[0], w_buf[0], preferred_element_type=jnp.float32
            )

            @pl.when(kk == 0)
            def _():
                acc[...] = prod0

            @pl.when(kk > 0)
            def _():
                acc[...] += prod0

            @pl.when(k0 + 2 < n_k)
            def _():
                w_copy(k0 + 2, d0, 0).start()

            xf32_copy(k1).wait()

            @pl.when(k1 >= 2)
            def _():
                wb_copy(k1 - 2, 1).wait()

            x_buf[1] = x_stage[...].astype(jnp.bfloat16)

            @pl.when(k1 + 1 < n_k)
            def _():
                xf32_copy(k1 + 1).start()

            wb_copy(k1, 1).start()
            wait_w_convert(k1, d0, 1)
            acc[...] += jnp.dot(
                x_buf[1], w_buf[1], preferred_element_type=jnp.float32
            )

            @pl.when(k1 + 2 < n_k)
            def _():
                w_copy(k1 + 2, d0, 1).start()

            return 0

        lax.fori_loop(0, n_k // 2, kbody0, 0)

        wb_copy(n_k - 2, 0).wait()
        wb_copy(n_k - 1, 1).wait()
        d1 = (my + offsets[1]) % N_DEV
        xbf_copy(0, 0).start()
        w_copy(0, d1, 0).start()
        xbf_copy(1, 1).start()
        w_copy(1, d1, 1).start()

        send_buf[0] = jnp.maximum(acc[...], 0.0).astype(jnp.bfloat16)
        rdma = pltpu.make_async_remote_copy(
            src_ref=send_buf.at[0],
            dst_ref=recv_buf.at[0],
            send_sem=send_sems.at[0],
            recv_sem=recv_sems.at[0],
            device_id=(d0,),
            device_id_type=pl.DeviceIdType.MESH,
        )
        rdma.start()
        send_rdmas.append(rdma)

        for s in range(1, N_DEV):
            d = (my + offsets[s]) % N_DEV

            def kbody(kk, _, d=d):
                k0 = 2 * kk
                k1 = k0 + 1
                xbf_copy(k0, 0).wait()
                wait_w_convert(k0, d, 0)
                prod0 = jnp.dot(
                    x_buf[0], w_buf[0], preferred_element_type=jnp.float32
                )

                @pl.when(kk == 0)
                def _():
                    acc[...] = prod0

                @pl.when(kk > 0)
                def _():
                    acc[...] += prod0

                @pl.when(k0 + 2 < n_k)
                def _():
                    xbf_copy(k0 + 2, 0).start()
                    w_copy(k0 + 2, d, 0).start()

                xbf_copy(k1, 1).wait()
                wait_w_convert(k1, d, 1)
                acc[...] += jnp.dot(
                    x_buf[1], w_buf[1], preferred_element_type=jnp.float32
                )

                @pl.when(k1 + 2 < n_k)
                def _():
                    xbf_copy(k1 + 2, 1).start()
                    w_copy(k1 + 2, d, 1).start()

                return 0

            lax.fori_loop(0, n_k // 2, kbody, 0)

            if s + 1 < N_DEV:
                d_next = (my + offsets[s + 1]) % N_DEV
                xbf_copy(0, 0).start()
                w_copy(0, d_next, 0).start()
                xbf_copy(1, 1).start()
                w_copy(1, d_next, 1).start()

            off = offsets[s]
            if off == 0:
                acc[...] = jnp.maximum(acc[...], 0.0)
                own_dma = pltpu.make_async_copy(
                    acc,
                    out_hbm.at[pl.ds(my * M_PER, M_PER), :],
                    out_sems.at[0],
                )
                own_dma.start()
            else:
                t = s
                sslot = s % 2
                if s == 2:
                    send_rdmas[0].wait_send()
                send_buf[sslot] = jnp.maximum(acc[...], 0.0).astype(
                    jnp.bfloat16
                )
                rdma = pltpu.make_async_remote_copy(
                    src_ref=send_buf.at[sslot],
                    dst_ref=recv_buf.at[t],
                    send_sem=send_sems.at[sslot],
                    recv_sem=recv_sems.at[t],
                    device_id=(d,),
                    device_id_type=pl.DeviceIdType.MESH,
                )
                rdma.start()
                send_rdmas.append(rdma)

        for t in range(N_DEV - 1):
            recv = pltpu.make_async_remote_copy(
                src_ref=send_buf.at[t % 2],
                dst_ref=recv_buf.at[t],
                send_sem=send_sems.at[t % 2],
                recv_sem=recv_sems.at[t],
                device_id=((my - t - 1) % N_DEV,),
                device_id_type=pl.DeviceIdType.MESH,
            )
            recv.wait_recv()
            own_dma.wait() if t == 0 else prev_cp.wait()
            j = (my - t - 1) % N_DEV
            acc[...] = recv_buf[t].astype(jnp.float32)
            prev_cp = pltpu.make_async_copy(
                acc,
                out_hbm.at[pl.ds(j * M_PER, M_PER), :],
                out_sems.at[(t + 1) % 2],
            )
            prev_cp.start()
        prev_cp.wait()
        for rdma in send_rdmas[1:]:
            rdma.wait_send()

        for p in range(1, N_DEV):
            pl.semaphore_signal(
                exit_sem, inc=1,
                device_id=((my + p) % N_DEV,),
                device_id_type=pl.DeviceIdType.MESH,
            )
        pl.semaphore_wait(exit_sem, N_DEV - 1)

    out, _xbf = pl.pallas_call(
        body,
        out_shape=(
            jax.ShapeDtypeStruct((N_DEV * m_per, n_blk), jnp.float32),
            jax.ShapeDtypeStruct((m_per, k_dim), jnp.bfloat16),
        ),
        in_specs=[
            pl.BlockSpec(memory_space=pl.ANY),
            pl.BlockSpec(memory_space=pl.ANY),
        ],
        out_specs=(
            pl.BlockSpec(memory_space=pl.ANY),
            pl.BlockSpec(memory_space=pl.ANY),
        ),
        scratch_shapes=[
            pltpu.VMEM((2, m_per, K_CHUNK), jnp.bfloat16),
            pltpu.VMEM((m_per, K_CHUNK), jnp.float32),
            pltpu.VMEM((2, K_CHUNK, n_blk), jnp.float32),
            pltpu.VMEM((2, K_CHUNK, n_blk), jnp.bfloat16),
            pltpu.VMEM((m_per, n_blk), jnp.float32),
            pltpu.VMEM((2, m_per, n_blk), jnp.bfloat16),
            pltpu.VMEM((N_DEV - 1, m_per, n_blk), jnp.bfloat16),
            pltpu.SemaphoreType.DMA((2,)),
            pltpu.SemaphoreType.DMA,
            pltpu.SemaphoreType.DMA((2,)),
            pltpu.SemaphoreType.DMA((2,)),
            pltpu.SemaphoreType.DMA((2,)),
            pltpu.SemaphoreType.DMA((2,)),
            pltpu.SemaphoreType.DMA((N_DEV - 1,)),
            pltpu.SemaphoreType.REGULAR,
        ],
        compiler_params=pltpu.CompilerParams(
            collective_id=0,
            vmem_limit_bytes=100 * 1024 * 1024,
        ),
    )(x, w_mat)
    return out
